# baseline (speedup 1.0000x reference)
import functools

import jax
import jax.numpy as jnp
from jax import lax
from jax.experimental import pallas as pl
from jax.experimental.pallas import tpu as pltpu
from jax.experimental.pallas import tpu_sc as plsc

BATCH = 4096
HIST = 200
EMBED_DIM = 32
VOCAB = 1000000

NUM_CORES = 2
NUM_SUBCORES = 16
NUM_WORKERS = NUM_CORES * NUM_SUBCORES

TOTAL = BATCH * HIST
PER_WORKER = TOTAL // NUM_WORKERS

PANEL = 128
NPANEL_FULL = VOCAB // PANEL
PANEL_ITERS = NPANEL_FULL // NUM_WORKERS + 1

_mesh = plsc.VectorSubcoreMesh(core_axis_name="c", subcore_axis_name="s")


def _transpose_panel(pan_v, tpan_v, rows):
    iota = lax.iota(jnp.int32, 16)
    for r in range(rows):
        for k in range(8):
            rowsel = iota + 16 * (k % 2)
            colsel = jnp.full((16,), 4 * r + k // 2, jnp.int32)
            vec = plsc.load_gather(pan_v, [rowsel, colsel])
            tpan_v[r, pl.ds(16 * k, 16)] = vec


@functools.partial(
    pl.kernel,
    mesh=_mesh,
    out_type=jax.ShapeDtypeStruct((VOCAB * EMBED_DIM // 128, 128), jnp.float32),
    scratch_types=[
        pltpu.VMEM((EMBED_DIM, PANEL), jnp.float32),
        pltpu.VMEM((EMBED_DIM, PANEL), jnp.float32),
    ],
    compiler_params=pltpu.CompilerParams(
        use_tc_tiling_on_sc=True, needs_layout_passes=False
    ),
)
def _transpose_pack(wt_hbm, wtail_hbm, wpk_hbm, pan_v, tpan_v):
    wid = lax.axis_index("s") * NUM_CORES + lax.axis_index("c")

    def body(i, carry):
        p = i * NUM_WORKERS + wid

        @pl.when(p < NPANEL_FULL)
        def _full():
            col0 = pl.multiple_of(p * PANEL, PANEL)
            pltpu.sync_copy(wt_hbm.at[:, pl.ds(col0, PANEL)], pan_v)
            _transpose_panel(pan_v, tpan_v, EMBED_DIM)
            row0 = pl.multiple_of(p * EMBED_DIM, EMBED_DIM)
            pltpu.sync_copy(tpan_v, wpk_hbm.at[pl.ds(row0, EMBED_DIM)])

        @pl.when(p == NPANEL_FULL)
        def _tail():
            pltpu.sync_copy(wtail_hbm, tpan_v.at[pl.ds(0, 16)])
            pltpu.sync_copy(
                tpan_v.at[pl.ds(0, 16)],
                wpk_hbm.at[pl.ds(NPANEL_FULL * EMBED_DIM, 16)],
            )

        return carry

    lax.fori_loop(0, PANEL_ITERS, body, 0)


CHUNK = 1600
NUM_CHUNKS = PER_WORKER // CHUNK


@functools.partial(
    pl.kernel,
    mesh=_mesh,
    out_type=jax.ShapeDtypeStruct((TOTAL, EMBED_DIM), jnp.float32),
    scratch_types=[
        pltpu.VMEM((CHUNK,), jnp.int32),
        pltpu.VMEM((CHUNK, EMBED_DIM), jnp.float32),
        pltpu.SemaphoreType.DMA,
    ],
    compiler_params=pltpu.CompilerParams(use_tc_tiling_on_sc=False),
)
def _embed_gather(table_hbm, idx_hbm, out_hbm, idx_v, rows_v, sem):
    wid = lax.axis_index("s") * NUM_CORES + lax.axis_index("c")
    base = wid * PER_WORKER

    def body(g, carry):
        off = pl.multiple_of(base + g * CHUNK, CHUNK)
        pltpu.sync_copy(idx_hbm.at[pl.ds(off, CHUNK)], idx_v)
        pltpu.async_copy(table_hbm.at[idx_v], rows_v, sem).wait()
        pltpu.sync_copy(rows_v, out_hbm.at[pl.ds(off, CHUNK)])
        return carry

    lax.fori_loop(0, NUM_CHUNKS, body, 0)


def kernel(doc, embed_weight):
    wtail = embed_weight[NPANEL_FULL * PANEL :].reshape(16, 128)
    wpk = _transpose_pack(embed_weight.T, wtail)
    wp = wpk.reshape(VOCAB, EMBED_DIM)
    idx = doc.reshape(-1).astype(jnp.int32)
    out = _embed_gather(wp, idx)
    return out.reshape(BATCH, HIST, EMBED_DIM)


# speedup vs baseline: 1.0971x; 1.0971x over previous
import functools

import jax
import jax.numpy as jnp
from jax import lax
from jax.experimental import pallas as pl
from jax.experimental.pallas import tpu as pltpu
from jax.experimental.pallas import tpu_sc as plsc

BATCH = 4096
HIST = 200
EMBED_DIM = 32
VOCAB = 1000000

NUM_CORES = 2
NUM_SUBCORES = 16
NUM_WORKERS = NUM_CORES * NUM_SUBCORES

TOTAL = BATCH * HIST
PER_WORKER = TOTAL // NUM_WORKERS

PW = 256
ROWS_OUT = PW * EMBED_DIM // 128
PANELS_MAIN = 3904
PER_W_PANELS = PANELS_MAIN // NUM_WORKERS
HALF_ITERS = PER_W_PANELS // 2
TAIL_VOCAB = VOCAB - PANELS_MAIN * PW
TAIL_ROWS = TAIL_VOCAB * EMBED_DIM // 128
PACKED_ROWS = VOCAB * EMBED_DIM // 128

_mesh = plsc.VectorSubcoreMesh(core_axis_name="c", subcore_axis_name="s")


@functools.partial(
    pl.kernel,
    mesh=_mesh,
    out_type=jax.ShapeDtypeStruct((PACKED_ROWS, 128), jnp.float32),
    scratch_types=[
        pltpu.VMEM((EMBED_DIM, PW), jnp.float32),
        pltpu.VMEM((EMBED_DIM, PW), jnp.float32),
        pltpu.VMEM((ROWS_OUT, 128), jnp.float32),
        pltpu.VMEM((ROWS_OUT, 128), jnp.float32),
        pltpu.VMEM((TAIL_ROWS, 128), jnp.float32),
        pltpu.SemaphoreType.DMA,
        pltpu.SemaphoreType.DMA,
        pltpu.SemaphoreType.DMA,
        pltpu.SemaphoreType.DMA,
    ],
    compiler_params=pltpu.CompilerParams(
        use_tc_tiling_on_sc=True, needs_layout_passes=False
    ),
)
def _transpose_pack(
    wt_hbm, wtail_hbm, wpk_hbm,
    pan0, pan1, tpan0, tpan1, tailbuf, si0, si1, so0, so1,
):
    wid = lax.axis_index("s") * NUM_CORES + lax.axis_index("c")
    base = wid * PER_W_PANELS

    def start_in(p, pan, sem):
        col0 = pl.multiple_of(p * PW, PW)
        pltpu.async_copy(wt_hbm.at[:, pl.ds(col0, PW)], pan, sem)

    def wait_in(pan, sem):
        pltpu.make_async_copy(wt_hbm.at[:, pl.ds(0, PW)], pan, sem).wait()

    def start_out(p, tpan, sem):
        row0 = pl.multiple_of(p * ROWS_OUT, ROWS_OUT)
        pltpu.async_copy(tpan, wpk_hbm.at[pl.ds(row0, ROWS_OUT)], sem)

    def wait_out(tpan, sem):
        pltpu.make_async_copy(tpan, wpk_hbm.at[pl.ds(0, ROWS_OUT)], sem).wait()

    def transpose(pan, tpan):
        iota = lax.iota(jnp.int32, 16)
        for r in range(ROWS_OUT):
            for k in range(8):
                vec = plsc.load_gather(
                    pan,
                    [iota + 16 * (k % 2), jnp.full((16,), 4 * r + k // 2, jnp.int32)],
                )
                tpan[r, pl.ds(16 * k, 16)] = vec

    start_in(base, pan0, si0)

    def body(j, carry):
        i0 = base + 2 * j
        i1 = i0 + 1
        start_in(i1, pan1, si1)
        wait_in(pan0, si0)

        @pl.when(j > 0)
        def _():
            wait_out(tpan0, so0)

        transpose(pan0, tpan0)
        start_out(i0, tpan0, so0)

        @pl.when(j < HALF_ITERS - 1)
        def _():
            start_in(i0 + 2, pan0, si0)

        wait_in(pan1, si1)

        @pl.when(j > 0)
        def _():
            wait_out(tpan1, so1)

        transpose(pan1, tpan1)
        start_out(i1, tpan1, so1)
        return carry

    lax.fori_loop(0, HALF_ITERS, body, 0)
    wait_out(tpan0, so0)
    wait_out(tpan1, so1)

    @pl.when(wid == 0)
    def _tail():
        pltpu.sync_copy(wtail_hbm, tailbuf)
        pltpu.sync_copy(tailbuf, wpk_hbm.at[pl.ds(PACKED_ROWS - TAIL_ROWS, TAIL_ROWS)])


CHUNK = 1600
NUM_CHUNKS = PER_WORKER // CHUNK


@functools.partial(
    pl.kernel,
    mesh=_mesh,
    out_type=jax.ShapeDtypeStruct((TOTAL, EMBED_DIM), jnp.float32),
    scratch_types=[
        pltpu.VMEM((CHUNK,), jnp.int32),
        pltpu.VMEM((CHUNK, EMBED_DIM), jnp.float32),
        pltpu.SemaphoreType.DMA,
    ],
    compiler_params=pltpu.CompilerParams(use_tc_tiling_on_sc=False),
)
def _embed_gather(table_hbm, idx_hbm, out_hbm, idx_v, rows_v, sem):
    wid = lax.axis_index("s") * NUM_CORES + lax.axis_index("c")
    base = wid * PER_WORKER

    def body(g, carry):
        off = pl.multiple_of(base + g * CHUNK, CHUNK)
        pltpu.sync_copy(idx_hbm.at[pl.ds(off, CHUNK)], idx_v)
        pltpu.async_copy(table_hbm.at[idx_v], rows_v, sem).wait()
        pltpu.sync_copy(rows_v, out_hbm.at[pl.ds(off, CHUNK)])
        return carry

    lax.fori_loop(0, NUM_CHUNKS, body, 0)


def kernel(doc, embed_weight):
    wtail = embed_weight[PANELS_MAIN * PW :].reshape(TAIL_ROWS, 128)
    wpk = _transpose_pack(embed_weight.T, wtail)
    wp = wpk.reshape(VOCAB, EMBED_DIM)
    idx = doc.reshape(-1).astype(jnp.int32)
    out = _embed_gather(wp, idx)
    return out.reshape(BATCH, HIST, EMBED_DIM)


# speedup vs baseline: 1.4435x; 1.3158x over previous
import functools

import jax
import jax.numpy as jnp
from jax import lax
from jax.experimental import pallas as pl
from jax.experimental.pallas import tpu as pltpu
from jax.experimental.pallas import tpu_sc as plsc

BATCH = 4096
HIST = 200
EMBED_DIM = 32
VOCAB = 1000000

NUM_CORES = 2
NUM_SUBCORES = 16
NUM_WORKERS = NUM_CORES * NUM_SUBCORES

TOTAL = BATCH * HIST
PER_WORKER = TOTAL // NUM_WORKERS

PW = 256
ROWS_OUT = PW * EMBED_DIM // 128
PANELS_MAIN = 3904
PER_W_PANELS = PANELS_MAIN // NUM_WORKERS
HALF_ITERS = PER_W_PANELS // 2
TAIL_VOCAB = VOCAB - PANELS_MAIN * PW
TAIL_ROWS = TAIL_VOCAB * EMBED_DIM // 128
PACKED_ROWS = VOCAB * EMBED_DIM // 128

_mesh = plsc.VectorSubcoreMesh(core_axis_name="c", subcore_axis_name="s")


@functools.partial(
    pl.kernel,
    mesh=_mesh,
    out_type=jax.ShapeDtypeStruct((PACKED_ROWS, 128), jnp.float32),
    scratch_types=[
        pltpu.VMEM((EMBED_DIM, PW), jnp.float32),
        pltpu.VMEM((EMBED_DIM, PW), jnp.float32),
        pltpu.VMEM((ROWS_OUT, 128), jnp.float32),
        pltpu.VMEM((ROWS_OUT, 128), jnp.float32),
        pltpu.VMEM((TAIL_ROWS, 128), jnp.float32),
        pltpu.SemaphoreType.DMA,
        pltpu.SemaphoreType.DMA,
        pltpu.SemaphoreType.DMA,
        pltpu.SemaphoreType.DMA,
    ],
    compiler_params=pltpu.CompilerParams(
        use_tc_tiling_on_sc=True,
        needs_layout_passes=False,
        disable_bounds_checks=True,
    ),
)
def _transpose_pack(
    wt_hbm, wtail_hbm, wpk_hbm,
    pan0, pan1, tpan0, tpan1, tailbuf, si0, si1, so0, so1,
):
    wid = lax.axis_index("s") * NUM_CORES + lax.axis_index("c")
    base = wid * PER_W_PANELS

    def start_in(p, pan, sem):
        col0 = pl.multiple_of(p * PW, PW)
        pltpu.async_copy(wt_hbm.at[:, pl.ds(col0, PW)], pan, sem)

    def wait_in(pan, sem):
        pltpu.make_async_copy(wt_hbm.at[:, pl.ds(0, PW)], pan, sem).wait()

    def start_out(p, tpan, sem):
        row0 = pl.multiple_of(p * ROWS_OUT, ROWS_OUT)
        pltpu.async_copy(tpan, wpk_hbm.at[pl.ds(row0, ROWS_OUT)], sem)

    def wait_out(tpan, sem):
        pltpu.make_async_copy(tpan, wpk_hbm.at[pl.ds(0, ROWS_OUT)], sem).wait()

    iota = lax.iota(jnp.int32, 16)
    row_even = iota
    row_odd = iota + 16

    def transpose(pan, tpan):
        def trow(r, carry):
            r4 = r * 4
            vecs = [
                plsc.load_gather(
                    pan,
                    [row_odd if k % 2 else row_even,
                     jnp.full((16,), r4 + k // 2, jnp.int32)],
                )
                for k in range(8)
            ]
            rowsel = jnp.full((16,), r, jnp.int32)
            for k in range(8):
                plsc.store_scatter(tpan, [rowsel, iota + 16 * k], vecs[k])
            return carry

        lax.fori_loop(0, ROWS_OUT, trow, 0)

    start_in(base, pan0, si0)

    def body(j, carry):
        i0 = base + 2 * j
        i1 = i0 + 1
        start_in(i1, pan1, si1)
        wait_in(pan0, si0)

        @pl.when(j > 0)
        def _():
            wait_out(tpan0, so0)

        transpose(pan0, tpan0)
        start_out(i0, tpan0, so0)

        @pl.when(j < HALF_ITERS - 1)
        def _():
            start_in(i0 + 2, pan0, si0)

        wait_in(pan1, si1)

        @pl.when(j > 0)
        def _():
            wait_out(tpan1, so1)

        transpose(pan1, tpan1)
        start_out(i1, tpan1, so1)
        return carry

    lax.fori_loop(0, HALF_ITERS, body, 0)
    wait_out(tpan0, so0)
    wait_out(tpan1, so1)

    @pl.when(wid == 0)
    def _tail():
        pltpu.sync_copy(wtail_hbm, tailbuf)
        pltpu.sync_copy(tailbuf, wpk_hbm.at[pl.ds(PACKED_ROWS - TAIL_ROWS, TAIL_ROWS)])


CHUNK = 1600
NUM_CHUNKS = PER_WORKER // CHUNK


@functools.partial(
    pl.kernel,
    mesh=_mesh,
    out_type=jax.ShapeDtypeStruct((TOTAL, EMBED_DIM), jnp.float32),
    scratch_types=[
        pltpu.VMEM((CHUNK,), jnp.int32),
        pltpu.VMEM((CHUNK, EMBED_DIM), jnp.float32),
        pltpu.SemaphoreType.DMA,
    ],
    compiler_params=pltpu.CompilerParams(use_tc_tiling_on_sc=False),
)
def _embed_gather(table_hbm, idx_hbm, out_hbm, idx_v, rows_v, sem):
    wid = lax.axis_index("s") * NUM_CORES + lax.axis_index("c")
    base = wid * PER_WORKER

    def body(g, carry):
        off = pl.multiple_of(base + g * CHUNK, CHUNK)
        pltpu.sync_copy(idx_hbm.at[pl.ds(off, CHUNK)], idx_v)
        pltpu.async_copy(table_hbm.at[idx_v], rows_v, sem).wait()
        pltpu.sync_copy(rows_v, out_hbm.at[pl.ds(off, CHUNK)])
        return carry

    lax.fori_loop(0, NUM_CHUNKS, body, 0)


def kernel(doc, embed_weight):
    wtail = embed_weight[PANELS_MAIN * PW :].reshape(TAIL_ROWS, 128)
    wpk = _transpose_pack(embed_weight.T, wtail)
    wp = wpk.reshape(VOCAB, EMBED_DIM)
    idx = doc.reshape(-1).astype(jnp.int32)
    out = _embed_gather(wp, idx)
    return out.reshape(BATCH, HIST, EMBED_DIM)


# speedup vs baseline: 2.0552x; 1.4237x over previous
import functools

import jax
import jax.numpy as jnp
from jax import lax
from jax.experimental import pallas as pl
from jax.experimental.pallas import tpu as pltpu
from jax.experimental.pallas import tpu_sc as plsc

BATCH = 4096
HIST = 200
EMBED_DIM = 32
VOCAB = 1000000

NUM_CORES = 2
NUM_SUBCORES = 16
NUM_WORKERS = NUM_CORES * NUM_SUBCORES

TOTAL = BATCH * HIST
PER_WORKER = TOTAL // NUM_WORKERS

PW = 256
ROWS_OUT = PW * EMBED_DIM // 128
PANELS_MAIN = 3904
PER_W_PANELS = PANELS_MAIN // NUM_WORKERS
HALF_ITERS = PER_W_PANELS // 2
TAIL_VOCAB = VOCAB - PANELS_MAIN * PW
TAIL_ROWS = TAIL_VOCAB * EMBED_DIM // 128
PACKED_ROWS = VOCAB * EMBED_DIM // 128

_mesh = plsc.VectorSubcoreMesh(core_axis_name="c", subcore_axis_name="s")


@functools.partial(
    pl.kernel,
    mesh=_mesh,
    out_type=jax.ShapeDtypeStruct((PACKED_ROWS, 128), jnp.float32),
    scratch_types=[
        pltpu.VMEM((EMBED_DIM, PW), jnp.float32),
        pltpu.VMEM((EMBED_DIM, PW), jnp.float32),
        pltpu.VMEM((ROWS_OUT, 128), jnp.float32),
        pltpu.VMEM((ROWS_OUT, 128), jnp.float32),
        pltpu.VMEM((TAIL_ROWS, 128), jnp.float32),
        pltpu.SemaphoreType.DMA,
        pltpu.SemaphoreType.DMA,
        pltpu.SemaphoreType.DMA,
        pltpu.SemaphoreType.DMA,
    ],
    compiler_params=pltpu.CompilerParams(
        use_tc_tiling_on_sc=True,
        needs_layout_passes=False,
        disable_bounds_checks=True,
    ),
)
def _transpose_pack(
    wt_hbm, wtail_hbm, wpk_hbm,
    pan0, pan1, tpan0, tpan1, tailbuf, si0, si1, so0, so1,
):
    wid = lax.axis_index("s") * NUM_CORES + lax.axis_index("c")
    base = wid * PER_W_PANELS

    def start_in(p, pan, sem):
        col0 = pl.multiple_of(p * PW, PW)
        pltpu.async_copy(wt_hbm.at[:, pl.ds(col0, PW)], pan, sem)

    def wait_in(pan, sem):
        pltpu.make_async_copy(wt_hbm.at[:, pl.ds(0, PW)], pan, sem).wait()

    def start_out(p, tpan, sem):
        row0 = pl.multiple_of(p * ROWS_OUT, ROWS_OUT)
        pltpu.async_copy(tpan, wpk_hbm.at[pl.ds(row0, ROWS_OUT)], sem)

    def wait_out(tpan, sem):
        pltpu.make_async_copy(tpan, wpk_hbm.at[pl.ds(0, ROWS_OUT)], sem).wait()

    iota = lax.iota(jnp.int32, 16)
    row_even = iota
    row_odd = iota + 16

    def transpose(pan, tpan):
        @plsc.parallel_loop(0, ROWS_OUT, unroll=4)
        def trow(r):
            r4 = r * 4
            vecs = [
                plsc.load_gather(
                    pan,
                    [row_odd if k % 2 else row_even,
                     jnp.full((16,), r4 + k // 2, jnp.int32)],
                )
                for k in range(8)
            ]
            rowsel = jnp.full((16,), r, jnp.int32)
            for k in range(8):
                plsc.store_scatter(tpan, [rowsel, iota + 16 * k], vecs[k])

    start_in(base, pan0, si0)

    def body(j, carry):
        i0 = base + 2 * j
        i1 = i0 + 1
        start_in(i1, pan1, si1)
        wait_in(pan0, si0)

        @pl.when(j > 0)
        def _():
            wait_out(tpan0, so0)

        transpose(pan0, tpan0)
        start_out(i0, tpan0, so0)

        @pl.when(j < HALF_ITERS - 1)
        def _():
            start_in(i0 + 2, pan0, si0)

        wait_in(pan1, si1)

        @pl.when(j > 0)
        def _():
            wait_out(tpan1, so1)

        transpose(pan1, tpan1)
        start_out(i1, tpan1, so1)
        return carry

    lax.fori_loop(0, HALF_ITERS, body, 0)
    wait_out(tpan0, so0)
    wait_out(tpan1, so1)

    @pl.when(wid == 0)
    def _tail():
        pltpu.sync_copy(wtail_hbm, tailbuf)
        pltpu.sync_copy(tailbuf, wpk_hbm.at[pl.ds(PACKED_ROWS - TAIL_ROWS, TAIL_ROWS)])


CHUNK = 1600
NUM_CHUNKS = PER_WORKER // CHUNK


@functools.partial(
    pl.kernel,
    mesh=_mesh,
    out_type=jax.ShapeDtypeStruct((TOTAL, 128), jnp.float32),
    scratch_types=[
        pltpu.VMEM((CHUNK,), jnp.int32),
        pltpu.VMEM((CHUNK, EMBED_DIM), jnp.float32),
        pltpu.SemaphoreType.DMA,
    ],
    compiler_params=pltpu.CompilerParams(use_tc_tiling_on_sc=False),
)
def _embed_gather(table_hbm, idx_hbm, out_hbm, idx_v, rows_v, sem):
    wid = lax.axis_index("s") * NUM_CORES + lax.axis_index("c")
    base = wid * PER_WORKER

    def body(g, carry):
        off = pl.multiple_of(base + g * CHUNK, CHUNK)
        pltpu.sync_copy(idx_hbm.at[pl.ds(off, CHUNK)], idx_v)
        pltpu.async_copy(table_hbm.at[idx_v], rows_v, sem).wait()
        pltpu.sync_copy(
            rows_v, out_hbm.at[pl.ds(off, CHUNK), pl.ds(0, EMBED_DIM)]
        )
        return carry

    lax.fori_loop(0, NUM_CHUNKS, body, 0)


def kernel(doc, embed_weight):
    wtail = embed_weight[PANELS_MAIN * PW :].reshape(TAIL_ROWS, 128)
    wpk = _transpose_pack(embed_weight.T, wtail)
    wp = wpk.reshape(VOCAB, EMBED_DIM)
    idx = doc.reshape(-1).astype(jnp.int32)
    wide = _embed_gather(wp, idx)
    return wide[:, :EMBED_DIM].reshape(BATCH, HIST, EMBED_DIM)


# speedup vs baseline: 2.5314x; 1.2317x over previous
import functools

import jax
import jax.numpy as jnp
from jax import lax
from jax.experimental import pallas as pl
from jax.experimental.pallas import tpu as pltpu
from jax.experimental.pallas import tpu_sc as plsc

BATCH = 4096
HIST = 200
EMBED_DIM = 32
VOCAB = 1000000

NUM_CORES = 2
NUM_SUBCORES = 16
NUM_WORKERS = NUM_CORES * NUM_SUBCORES

TOTAL = BATCH * HIST
PER_WORKER = TOTAL // NUM_WORKERS

PW = 256
ROWS_OUT = PW * EMBED_DIM // 128
PANELS_MAIN = 3904
PER_W_PANELS = PANELS_MAIN // NUM_WORKERS
HALF_ITERS = PER_W_PANELS // 2
TAIL_VOCAB = VOCAB - PANELS_MAIN * PW
TAIL_ROWS = TAIL_VOCAB * EMBED_DIM // 128
PACKED_ROWS = VOCAB * EMBED_DIM // 128

_mesh = plsc.VectorSubcoreMesh(core_axis_name="c", subcore_axis_name="s")


@functools.partial(
    pl.kernel,
    mesh=_mesh,
    out_type=jax.ShapeDtypeStruct((PACKED_ROWS, 128), jnp.float32),
    scratch_types=[
        pltpu.VMEM((EMBED_DIM, PW), jnp.float32),
        pltpu.VMEM((EMBED_DIM, PW), jnp.float32),
        pltpu.VMEM((ROWS_OUT, 128), jnp.float32),
        pltpu.VMEM((ROWS_OUT, 128), jnp.float32),
        pltpu.VMEM((TAIL_ROWS, 128), jnp.float32),
        pltpu.SemaphoreType.DMA,
        pltpu.SemaphoreType.DMA,
        pltpu.SemaphoreType.DMA,
        pltpu.SemaphoreType.DMA,
    ],
    compiler_params=pltpu.CompilerParams(
        use_tc_tiling_on_sc=True,
        needs_layout_passes=False,
        disable_bounds_checks=True,
    ),
)
def _transpose_pack(
    wt_hbm, wtail_hbm, wpk_hbm,
    pan0, pan1, tpan0, tpan1, tailbuf, si0, si1, so0, so1,
):
    wid = lax.axis_index("s") * NUM_CORES + lax.axis_index("c")
    base = wid * PER_W_PANELS

    def start_in(p, pan, sem):
        col0 = pl.multiple_of(p * PW, PW)
        pltpu.async_copy(wt_hbm.at[:, pl.ds(col0, PW)], pan, sem)

    def wait_in(pan, sem):
        pltpu.make_async_copy(wt_hbm.at[:, pl.ds(0, PW)], pan, sem).wait()

    def start_out(p, tpan, sem):
        row0 = pl.multiple_of(p * ROWS_OUT, ROWS_OUT)
        pltpu.async_copy(tpan, wpk_hbm.at[pl.ds(row0, ROWS_OUT)], sem)

    def wait_out(tpan, sem):
        pltpu.make_async_copy(tpan, wpk_hbm.at[pl.ds(0, ROWS_OUT)], sem).wait()

    iota = lax.iota(jnp.int32, 16)
    cvecs = [(iota + s) % 16 for s in range(16)]
    dvecs = [cvecs[s] * EMBED_DIM + iota for s in range(16)]

    def transpose(pan, tpan):
        @plsc.parallel_loop(0, PW // 16, unroll=2)
        def tblock(cb):
            c0 = cb * 16
            for d0 in (0, 16):
                rowsel = iota + d0
                base2 = c0 * EMBED_DIM + d0
                for s in range(16):
                    vec = plsc.load_gather(pan, [rowsel, cvecs[s] + c0])
                    dstflat = dvecs[s] + base2
                    plsc.store_scatter(
                        tpan,
                        [lax.shift_right_logical(dstflat, 7), dstflat & 127],
                        vec,
                    )

    start_in(base, pan0, si0)

    def body(j, carry):
        i0 = base + 2 * j
        i1 = i0 + 1
        start_in(i1, pan1, si1)
        wait_in(pan0, si0)

        @pl.when(j > 0)
        def _():
            wait_out(tpan0, so0)

        transpose(pan0, tpan0)
        start_out(i0, tpan0, so0)

        @pl.when(j < HALF_ITERS - 1)
        def _():
            start_in(i0 + 2, pan0, si0)

        wait_in(pan1, si1)

        @pl.when(j > 0)
        def _():
            wait_out(tpan1, so1)

        transpose(pan1, tpan1)
        start_out(i1, tpan1, so1)
        return carry

    lax.fori_loop(0, HALF_ITERS, body, 0)
    wait_out(tpan0, so0)
    wait_out(tpan1, so1)

    @pl.when(wid == 0)
    def _tail():
        pltpu.sync_copy(wtail_hbm, tailbuf)
        pltpu.sync_copy(tailbuf, wpk_hbm.at[pl.ds(PACKED_ROWS - TAIL_ROWS, TAIL_ROWS)])


CHUNK = 1600
NUM_CHUNKS = PER_WORKER // CHUNK


@functools.partial(
    pl.kernel,
    mesh=_mesh,
    out_type=jax.ShapeDtypeStruct((TOTAL, 128), jnp.float32),
    scratch_types=[
        pltpu.VMEM((CHUNK,), jnp.int32),
        pltpu.VMEM((CHUNK, EMBED_DIM), jnp.float32),
        pltpu.SemaphoreType.DMA,
    ],
    compiler_params=pltpu.CompilerParams(use_tc_tiling_on_sc=False),
)
def _embed_gather(table_hbm, idx_hbm, out_hbm, idx_v, rows_v, sem):
    wid = lax.axis_index("s") * NUM_CORES + lax.axis_index("c")
    base = wid * PER_WORKER

    def body(g, carry):
        off = pl.multiple_of(base + g * CHUNK, CHUNK)
        pltpu.sync_copy(idx_hbm.at[pl.ds(off, CHUNK)], idx_v)
        pltpu.async_copy(table_hbm.at[idx_v], rows_v, sem).wait()
        pltpu.sync_copy(
            rows_v, out_hbm.at[pl.ds(off, CHUNK), pl.ds(0, EMBED_DIM)]
        )
        return carry

    lax.fori_loop(0, NUM_CHUNKS, body, 0)


def kernel(doc, embed_weight):
    wtail = embed_weight[PANELS_MAIN * PW :].reshape(TAIL_ROWS, 128)
    wpk = _transpose_pack(embed_weight.T, wtail)
    wp = wpk.reshape(VOCAB, EMBED_DIM)
    idx = doc.reshape(-1).astype(jnp.int32)
    wide = _embed_gather(wp, idx)
    return wide[:, :EMBED_DIM].reshape(BATCH, HIST, EMBED_DIM)


# speedup vs baseline: 2.6221x; 1.0358x over previous
import functools

import jax
import jax.numpy as jnp
from jax import lax
from jax.experimental import pallas as pl
from jax.experimental.pallas import tpu as pltpu
from jax.experimental.pallas import tpu_sc as plsc

BATCH = 4096
HIST = 200
EMBED_DIM = 32
VOCAB = 1000000

NUM_CORES = 2
NUM_SUBCORES = 16
NUM_WORKERS = NUM_CORES * NUM_SUBCORES

TOTAL = BATCH * HIST
PER_WORKER = TOTAL // NUM_WORKERS

PW = 256
ROWS_OUT = PW * EMBED_DIM // 128
PANELS_MAIN = 3904
PER_W_PANELS = PANELS_MAIN // NUM_WORKERS
HALF_ITERS = PER_W_PANELS // 2
TAIL_VOCAB = VOCAB - PANELS_MAIN * PW
TAIL_ROWS = TAIL_VOCAB * EMBED_DIM // 128
PACKED_ROWS = VOCAB * EMBED_DIM // 128

_mesh = plsc.VectorSubcoreMesh(core_axis_name="c", subcore_axis_name="s")


@functools.partial(
    pl.kernel,
    mesh=_mesh,
    out_type=jax.ShapeDtypeStruct((VOCAB * EMBED_DIM,), jnp.float32),
    scratch_types=[
        pltpu.VMEM((EMBED_DIM, PW), jnp.float32),
        pltpu.VMEM((EMBED_DIM, PW), jnp.float32),
        pltpu.VMEM((PW * EMBED_DIM,), jnp.float32),
        pltpu.VMEM((PW * EMBED_DIM,), jnp.float32),
        pltpu.VMEM((TAIL_ROWS * 128,), jnp.float32),
        pltpu.SemaphoreType.DMA,
        pltpu.SemaphoreType.DMA,
        pltpu.SemaphoreType.DMA,
        pltpu.SemaphoreType.DMA,
    ],
    compiler_params=pltpu.CompilerParams(
        use_tc_tiling_on_sc=True,
        needs_layout_passes=False,
        disable_bounds_checks=True,
    ),
)
def _transpose_pack(
    wt_hbm, wtail_hbm, wpk_hbm,
    pan0, pan1, tpan0, tpan1, tailbuf, si0, si1, so0, so1,
):
    wid = lax.axis_index("s") * NUM_CORES + lax.axis_index("c")
    base = wid * PER_W_PANELS

    def start_in(p, pan, sem):
        col0 = pl.multiple_of(p * PW, PW)
        pltpu.async_copy(wt_hbm.at[:, pl.ds(col0, PW)], pan, sem)

    def wait_in(pan, sem):
        pltpu.make_async_copy(wt_hbm.at[:, pl.ds(0, PW)], pan, sem).wait()

    def start_out(p, tpan, sem):
        e0 = pl.multiple_of(p * (PW * EMBED_DIM), PW * EMBED_DIM)
        pltpu.async_copy(tpan, wpk_hbm.at[pl.ds(e0, PW * EMBED_DIM)], sem)

    def wait_out(tpan, sem):
        pltpu.make_async_copy(
            tpan, wpk_hbm.at[pl.ds(0, PW * EMBED_DIM)], sem
        ).wait()

    iota = lax.iota(jnp.int32, 16)
    cvecs = [(iota + s) % 16 for s in range(16)]
    dvecs = [cvecs[s] * EMBED_DIM + iota for s in range(16)]

    def transpose(pan, tpan):
        @plsc.parallel_loop(0, PW // 16, unroll=4)
        def tblock(cb):
            c0 = cb * 16
            for d0 in (0, 16):
                rowsel = iota + d0
                base2 = c0 * EMBED_DIM + d0
                for s in range(16):
                    vec = plsc.load_gather(pan, [rowsel, cvecs[s] + c0])
                    plsc.store_scatter(tpan, [dvecs[s] + base2], vec)

    start_in(base, pan0, si0)

    def body(j, carry):
        i0 = base + 2 * j
        i1 = i0 + 1
        start_in(i1, pan1, si1)
        wait_in(pan0, si0)

        @pl.when(j > 0)
        def _():
            wait_out(tpan0, so0)

        transpose(pan0, tpan0)
        start_out(i0, tpan0, so0)

        @pl.when(j < HALF_ITERS - 1)
        def _():
            start_in(i0 + 2, pan0, si0)

        wait_in(pan1, si1)

        @pl.when(j > 0)
        def _():
            wait_out(tpan1, so1)

        transpose(pan1, tpan1)
        start_out(i1, tpan1, so1)
        return carry

    lax.fori_loop(0, HALF_ITERS, body, 0)
    wait_out(tpan0, so0)
    wait_out(tpan1, so1)

    @pl.when(wid == 0)
    def _tail():
        pltpu.sync_copy(wtail_hbm, tailbuf)
        pltpu.sync_copy(
            tailbuf,
            wpk_hbm.at[pl.ds(VOCAB * EMBED_DIM - TAIL_ROWS * 128, TAIL_ROWS * 128)],
        )


CHUNK = 1600
NUM_CHUNKS = PER_WORKER // CHUNK


@functools.partial(
    pl.kernel,
    mesh=_mesh,
    out_type=jax.ShapeDtypeStruct((TOTAL, 128), jnp.float32),
    scratch_types=[
        pltpu.VMEM((CHUNK,), jnp.int32),
        pltpu.VMEM((CHUNK, EMBED_DIM), jnp.float32),
        pltpu.SemaphoreType.DMA,
    ],
    compiler_params=pltpu.CompilerParams(use_tc_tiling_on_sc=False),
)
def _embed_gather(table_hbm, idx_hbm, out_hbm, idx_v, rows_v, sem):
    wid = lax.axis_index("s") * NUM_CORES + lax.axis_index("c")
    base = wid * PER_WORKER

    def body(g, carry):
        off = pl.multiple_of(base + g * CHUNK, CHUNK)
        pltpu.sync_copy(idx_hbm.at[pl.ds(off, CHUNK)], idx_v)
        pltpu.async_copy(table_hbm.at[idx_v], rows_v, sem).wait()
        pltpu.sync_copy(
            rows_v, out_hbm.at[pl.ds(off, CHUNK), pl.ds(0, EMBED_DIM)]
        )
        return carry

    lax.fori_loop(0, NUM_CHUNKS, body, 0)


def kernel(doc, embed_weight):
    wtail = embed_weight[PANELS_MAIN * PW :].reshape(TAIL_ROWS * 128)
    wpk = _transpose_pack(embed_weight.T, wtail)
    wp = wpk.reshape(VOCAB, EMBED_DIM)
    idx = doc.reshape(-1).astype(jnp.int32)
    wide = _embed_gather(wp, idx)
    return wide[:, :EMBED_DIM].reshape(BATCH, HIST, EMBED_DIM)


# speedup vs baseline: 2.7138x; 1.0350x over previous
import functools

import jax
import jax.numpy as jnp
from jax import lax
from jax.experimental import pallas as pl
from jax.experimental.pallas import tpu as pltpu
from jax.experimental.pallas import tpu_sc as plsc

BATCH = 4096
HIST = 200
EMBED_DIM = 32
VOCAB = 1000000

NUM_CORES = 2
NUM_SUBCORES = 16
NUM_WORKERS = NUM_CORES * NUM_SUBCORES

TOTAL = BATCH * HIST
PER_WORKER = TOTAL // NUM_WORKERS

PW = 256
ROWS_OUT = PW * EMBED_DIM // 128
PANELS_MAIN = 3904
PER_W_PANELS = PANELS_MAIN // NUM_WORKERS
HALF_ITERS = PER_W_PANELS // 2
TAIL_VOCAB = VOCAB - PANELS_MAIN * PW
TAIL_ROWS = TAIL_VOCAB * EMBED_DIM // 128
PACKED_ROWS = VOCAB * EMBED_DIM // 128

_mesh = plsc.VectorSubcoreMesh(core_axis_name="c", subcore_axis_name="s")


@functools.partial(
    pl.kernel,
    mesh=_mesh,
    out_type=jax.ShapeDtypeStruct((VOCAB * EMBED_DIM,), jnp.float32),
    scratch_types=[
        pltpu.VMEM((EMBED_DIM, PW), jnp.float32),
        pltpu.VMEM((EMBED_DIM, PW), jnp.float32),
        pltpu.VMEM((PW * EMBED_DIM,), jnp.float32),
        pltpu.VMEM((PW * EMBED_DIM,), jnp.float32),
        pltpu.VMEM((TAIL_ROWS * 128,), jnp.float32),
        pltpu.SemaphoreType.DMA,
        pltpu.SemaphoreType.DMA,
        pltpu.SemaphoreType.DMA,
        pltpu.SemaphoreType.DMA,
    ],
    compiler_params=pltpu.CompilerParams(
        use_tc_tiling_on_sc=True,
        needs_layout_passes=False,
        disable_bounds_checks=True,
    ),
)
def _transpose_pack(
    wt_hbm, wtail_hbm, wpk_hbm,
    pan0, pan1, tpan0, tpan1, tailbuf, si0, si1, so0, so1,
):
    wid = lax.axis_index("s") * NUM_CORES + lax.axis_index("c")
    base = wid * PER_W_PANELS

    def start_in(p, pan, sem):
        col0 = pl.multiple_of(p * PW, PW)
        pltpu.async_copy(wt_hbm.at[:, pl.ds(col0, PW)], pan, sem)

    def wait_in(pan, sem):
        pltpu.make_async_copy(wt_hbm.at[:, pl.ds(0, PW)], pan, sem).wait()

    def start_out(p, tpan, sem):
        e0 = pl.multiple_of(p * (PW * EMBED_DIM), PW * EMBED_DIM)
        pltpu.async_copy(tpan, wpk_hbm.at[pl.ds(e0, PW * EMBED_DIM)], sem)

    def wait_out(tpan, sem):
        pltpu.make_async_copy(
            tpan, wpk_hbm.at[pl.ds(0, PW * EMBED_DIM)], sem
        ).wait()

    iota = lax.iota(jnp.int32, 16)
    cvecs = [(iota + s) % 16 for s in range(16)]
    dvecs = [cvecs[s] * EMBED_DIM + iota for s in range(16)]

    def transpose(pan, tpan):
        @plsc.parallel_loop(0, PW // 16, unroll=4)
        def tblock(cb):
            c0 = cb * 16
            for d0 in (0, 16):
                rowsel = iota + d0
                base2 = c0 * EMBED_DIM + d0
                for s in range(16):
                    vec = plsc.load_gather(pan, [rowsel, cvecs[s] + c0])
                    plsc.store_scatter(tpan, [dvecs[s] + base2], vec)

    start_in(base, pan0, si0)

    def body(j, carry):
        i0 = base + 2 * j
        i1 = i0 + 1
        start_in(i1, pan1, si1)
        wait_in(pan0, si0)

        @pl.when(j > 0)
        def _():
            wait_out(tpan0, so0)

        transpose(pan0, tpan0)
        start_out(i0, tpan0, so0)

        @pl.when(j < HALF_ITERS - 1)
        def _():
            start_in(i0 + 2, pan0, si0)

        wait_in(pan1, si1)

        @pl.when(j > 0)
        def _():
            wait_out(tpan1, so1)

        transpose(pan1, tpan1)
        start_out(i1, tpan1, so1)
        return carry

    lax.fori_loop(0, HALF_ITERS, body, 0)
    wait_out(tpan0, so0)
    wait_out(tpan1, so1)

    @pl.when(wid == 0)
    def _tail():
        pltpu.sync_copy(wtail_hbm, tailbuf)
        pltpu.sync_copy(
            tailbuf,
            wpk_hbm.at[pl.ds(VOCAB * EMBED_DIM - TAIL_ROWS * 128, TAIL_ROWS * 128)],
        )


CHUNK = 3200
NUM_CHUNKS = PER_WORKER // CHUNK


@functools.partial(
    pl.kernel,
    mesh=_mesh,
    out_type=jax.ShapeDtypeStruct((TOTAL, 128), jnp.float32),
    scratch_types=[
        pltpu.VMEM((CHUNK,), jnp.int32),
        pltpu.VMEM((CHUNK, EMBED_DIM), jnp.float32),
        pltpu.SemaphoreType.DMA,
    ],
    compiler_params=pltpu.CompilerParams(use_tc_tiling_on_sc=False),
)
def _embed_gather(table_hbm, idx_hbm, out_hbm, idx_v, rows_v, sem):
    wid = lax.axis_index("s") * NUM_CORES + lax.axis_index("c")
    base = wid * PER_WORKER

    def body(g, carry):
        off = pl.multiple_of(base + g * CHUNK, CHUNK)
        pltpu.sync_copy(idx_hbm.at[pl.ds(off, CHUNK)], idx_v)
        pltpu.async_copy(table_hbm.at[idx_v], rows_v, sem).wait()
        pltpu.sync_copy(
            rows_v, out_hbm.at[pl.ds(off, CHUNK), pl.ds(0, EMBED_DIM)]
        )
        return carry

    lax.fori_loop(0, NUM_CHUNKS, body, 0)


def kernel(doc, embed_weight):
    wtail = embed_weight[PANELS_MAIN * PW :].reshape(TAIL_ROWS * 128)
    wpk = _transpose_pack(embed_weight.T, wtail)
    wp = wpk.reshape(VOCAB, EMBED_DIM)
    idx = doc.reshape(-1).astype(jnp.int32)
    wide = _embed_gather(wp, idx)
    return wide[:, :EMBED_DIM].reshape(BATCH, HIST, EMBED_DIM)


# speedup vs baseline: 3.3010x; 1.2164x over previous
import functools

import jax
import jax.numpy as jnp
from jax import lax
from jax.experimental import pallas as pl
from jax.experimental.pallas import tpu as pltpu
from jax.experimental.pallas import tpu_sc as plsc

BATCH = 4096
HIST = 200
EMBED_DIM = 32
VOCAB = 1000000

NUM_CORES = 2
NUM_SUBCORES = 16
NUM_WORKERS = NUM_CORES * NUM_SUBCORES

TOTAL = BATCH * HIST
PER_WORKER = TOTAL // NUM_WORKERS

PW = 512
PANELS_MAIN = 1952
PER_W_PANELS = PANELS_MAIN // NUM_WORKERS
PAIR_ITERS = 30
P_ELEMS = PW * EMBED_DIM
TAIL_VOCAB = VOCAB - PANELS_MAIN * PW
TAIL_ROWS = TAIL_VOCAB * EMBED_DIM // 128

_mesh = plsc.VectorSubcoreMesh(core_axis_name="c", subcore_axis_name="s")


@functools.partial(
    pl.kernel,
    mesh=_mesh,
    out_type=jax.ShapeDtypeStruct((VOCAB * EMBED_DIM,), jnp.float32),
    scratch_types=[
        pltpu.VMEM((EMBED_DIM, PW), jnp.float32),
        pltpu.VMEM((EMBED_DIM, PW), jnp.float32),
        pltpu.VMEM((P_ELEMS,), jnp.float32),
        pltpu.VMEM((P_ELEMS,), jnp.float32),
        pltpu.VMEM((TAIL_ROWS * 128,), jnp.float32),
        pltpu.SemaphoreType.DMA,
        pltpu.SemaphoreType.DMA,
        pltpu.SemaphoreType.DMA,
        pltpu.SemaphoreType.DMA,
    ],
    compiler_params=pltpu.CompilerParams(
        use_tc_tiling_on_sc=True,
        needs_layout_passes=False,
        disable_bounds_checks=True,
    ),
)
def _transpose_pack(
    wt_hbm, wtail_hbm, wpk_hbm,
    pan0, pan1, tpan0, tpan1, tailbuf, si0, si1, so0, so1,
):
    wid = lax.axis_index("s") * NUM_CORES + lax.axis_index("c")
    base = wid * PER_W_PANELS

    def start_in(p, pan, sem):
        col0 = pl.multiple_of(p * PW, PW)
        pltpu.async_copy(wt_hbm.at[:, pl.ds(col0, PW)], pan, sem)

    def wait_in(pan, sem):
        pltpu.make_async_copy(wt_hbm.at[:, pl.ds(0, PW)], pan, sem).wait()

    def start_out(p, tpan, sem):
        e0 = pl.multiple_of(p * P_ELEMS, P_ELEMS)
        pltpu.async_copy(tpan, wpk_hbm.at[pl.ds(e0, P_ELEMS)], sem)

    def wait_out(tpan, sem):
        pltpu.make_async_copy(tpan, wpk_hbm.at[pl.ds(0, P_ELEMS)], sem).wait()

    iota = lax.iota(jnp.int32, 16)
    cvecs = [(iota + s) % 16 for s in range(16)]
    dvecs = [cvecs[s] * EMBED_DIM + iota for s in range(16)]

    def transpose(pan, tpan):
        @plsc.parallel_loop(0, PW // 16, unroll=4)
        def tblock(cb):
            c0 = cb * 16
            for d0 in (0, 16):
                rowsel = iota + d0
                base2 = c0 * EMBED_DIM + d0
                for s in range(16):
                    vec = plsc.load_gather(pan, [rowsel, cvecs[s] + c0])
                    plsc.store_scatter(tpan, [dvecs[s] + base2], vec)

    start_in(base, pan0, si0)

    def body(j, carry):
        i0 = base + 2 * j
        i1 = i0 + 1
        start_in(i1, pan1, si1)
        wait_in(pan0, si0)

        @pl.when(j > 0)
        def _():
            wait_out(tpan0, so0)

        transpose(pan0, tpan0)
        start_out(i0, tpan0, so0)

        start_in(i0 + 2, pan0, si0)
        wait_in(pan1, si1)

        @pl.when(j > 0)
        def _():
            wait_out(tpan1, so1)

        transpose(pan1, tpan1)
        start_out(i1, tpan1, so1)
        return carry

    lax.fori_loop(0, PAIR_ITERS, body, 0)
    wait_in(pan0, si0)
    wait_out(tpan0, so0)
    transpose(pan0, tpan0)
    start_out(base + PER_W_PANELS - 1, tpan0, so0)
    wait_out(tpan0, so0)
    wait_out(tpan1, so1)

    @pl.when(wid == 0)
    def _tail():
        pltpu.sync_copy(wtail_hbm, tailbuf)
        pltpu.sync_copy(
            tailbuf,
            wpk_hbm.at[pl.ds(VOCAB * EMBED_DIM - TAIL_ROWS * 128, TAIL_ROWS * 128)],
        )


CHUNK = 3200
NUM_CHUNKS = PER_WORKER // CHUNK


@functools.partial(
    pl.kernel,
    mesh=_mesh,
    out_type=jax.ShapeDtypeStruct((TOTAL, 128), jnp.float32),
    scratch_types=[
        pltpu.VMEM((CHUNK,), jnp.int32),
        pltpu.VMEM((CHUNK, EMBED_DIM), jnp.float32),
        pltpu.SemaphoreType.DMA,
    ],
    compiler_params=pltpu.CompilerParams(use_tc_tiling_on_sc=False),
)
def _embed_gather(table_hbm, idx_hbm, out_hbm, idx_v, rows_v, sem):
    wid = lax.axis_index("s") * NUM_CORES + lax.axis_index("c")
    base = wid * PER_WORKER

    def body(g, carry):
        off = pl.multiple_of(base + g * CHUNK, CHUNK)
        pltpu.sync_copy(idx_hbm.at[pl.ds(off, CHUNK)], idx_v)
        pltpu.async_copy(table_hbm.at[idx_v], rows_v, sem).wait()
        pltpu.sync_copy(
            rows_v, out_hbm.at[pl.ds(off, CHUNK), pl.ds(0, EMBED_DIM)]
        )
        return carry

    lax.fori_loop(0, NUM_CHUNKS, body, 0)


def kernel(doc, embed_weight):
    wtail = embed_weight[PANELS_MAIN * PW :].reshape(TAIL_ROWS * 128)
    wpk = _transpose_pack(embed_weight.T, wtail)
    wp = wpk.reshape(VOCAB, EMBED_DIM)
    idx = doc.reshape(-1).astype(jnp.int32)
    wide = _embed_gather(wp, idx)
    return wide[:, :EMBED_DIM].reshape(BATCH, HIST, EMBED_DIM)
